# baseline (speedup 1.0000x reference)
import functools

import jax
import jax.numpy as jnp
from jax import lax
from jax.experimental import pallas as pl
from jax.experimental.pallas import tpu as pltpu
from jax.experimental.pallas import tpu_sc as plsc

BATCH = 16384
EMBED_DIM = 64
PAD_DIM = 128
LANES = 16

_info = plsc.get_sparse_core_info()
_NC, _NS = _info.num_cores, _info.num_subcores
_NW = _NC * _NS
_B_PER_W = BATCH // _NW
_CHUNK = 128
_NCH = _B_PER_W // _CHUNK


def _lookup_body(labels_hbm, table_hbm, out_hbm, idx_v, g_v, t_v, gsem, wsem):
    wid = lax.axis_index("s") * _NC + lax.axis_index("c")
    base = wid * _B_PER_W
    pltpu.sync_copy(labels_hbm.at[pl.ds(base, _B_PER_W)], idx_v)

    lane = lax.iota(jnp.int32, LANES)

    def gather_chunk(t):
        return pltpu.async_copy(
            table_hbm.at[idx_v.at[pl.ds(t * _CHUNK, _CHUNK)]],
            g_v.at[t % 2],
            gsem[t % 2],
        )

    def transpose_chunk(t):
        def block(ii, _):
            row_idx = ii * LANES + lane
            for j in range(EMBED_DIM):
                col_idx = jnp.full((LANES,), j, jnp.int32)
                vals = plsc.load_gather(g_v.at[t % 2], [row_idx, col_idx])
                t_v[t % 2, j, pl.ds(ii * LANES, LANES)] = vals
            return 0

        lax.fori_loop(0, _CHUNK // LANES, block, 0)

    def write_chunk(t):
        return pltpu.async_copy(
            t_v.at[t % 2],
            out_hbm.at[:, pl.ds(base + t * _CHUNK, _CHUNK)],
            wsem[t % 2],
        )

    writes = [None, None]
    g = gather_chunk(0)
    for t in range(_NCH):
        g.wait()
        if t + 1 < _NCH:
            g_next = gather_chunk(t + 1)
        if writes[t % 2] is not None:
            writes[t % 2].wait()
        transpose_chunk(t)
        writes[t % 2] = write_chunk(t)
        if t + 1 < _NCH:
            g = g_next
    writes[(_NCH - 1) % 2].wait()
    writes[_NCH % 2].wait()


@jax.jit
def kernel(labels, table):
    table_pad = jnp.pad(table, ((0, 0), (0, PAD_DIM - EMBED_DIM)))
    k = functools.partial(
        pl.kernel,
        mesh=plsc.VectorSubcoreMesh(core_axis_name="c", subcore_axis_name="s"),
        out_type=jax.ShapeDtypeStruct((EMBED_DIM, BATCH), jnp.float32),
        scratch_types=[
            pltpu.VMEM((_B_PER_W,), jnp.int32),
            pltpu.VMEM((2, _CHUNK, PAD_DIM), jnp.float32),
            pltpu.VMEM((2, EMBED_DIM, _CHUNK), jnp.float32),
            [pltpu.SemaphoreType.DMA, pltpu.SemaphoreType.DMA],
            [pltpu.SemaphoreType.DMA, pltpu.SemaphoreType.DMA],
        ],
        compiler_params=pltpu.CompilerParams(
            use_tc_tiling_on_sc=True, needs_layout_passes=False
        ),
    )(_lookup_body)
    return k(labels, table_pad).T


# speedup vs baseline: 1.2723x; 1.2723x over previous
import functools

import jax
import jax.numpy as jnp
from jax import lax
from jax.experimental import pallas as pl
from jax.experimental.pallas import tpu as pltpu
from jax.experimental.pallas import tpu_sc as plsc

BATCH = 16384
EMBED_DIM = 64
PAD_DIM = 128
LANES = 16

_info = plsc.get_sparse_core_info()
_NC, _NS = _info.num_cores, _info.num_subcores
_NW = _NC * _NS
_B_PER_W = BATCH // _NW
_CHUNK = 128
_NCH = _B_PER_W // _CHUNK


def _lookup_body(labels_hbm, table_hbm, out_hbm, idx_v, g_v, t_v, gsem, wsem):
    wid = lax.axis_index("s") * _NC + lax.axis_index("c")
    base = wid * _B_PER_W
    pltpu.sync_copy(labels_hbm.at[pl.ds(base, _B_PER_W)], idx_v)

    lane = lax.iota(jnp.int32, LANES)

    def gather_chunk(t):
        return pltpu.async_copy(
            table_hbm.at[idx_v.at[pl.ds(t * _CHUNK, _CHUNK)]],
            g_v.at[t % 2],
            gsem[t % 2],
        )

    def transpose_chunk(t):
        nblk = _CHUNK // LANES

        @plsc.parallel_loop(0, EMBED_DIM * nblk, unroll=8)
        def _(v):
            ii = lax.bitwise_and(v, nblk - 1)
            j = lax.shift_right_logical(v, 3)
            row_idx = ii * LANES + lane
            col_idx = jnp.full((LANES,), j, jnp.int32)
            vals = plsc.load_gather(g_v.at[t % 2], [row_idx, col_idx])
            t_v[t % 2, j, pl.ds(ii * LANES, LANES)] = vals

    def write_chunk(t):
        return pltpu.async_copy(
            t_v.at[t % 2],
            out_hbm.at[:, pl.ds(base + t * _CHUNK, _CHUNK)],
            wsem[t % 2],
        )

    writes = [None, None]
    g = gather_chunk(0)
    for t in range(_NCH):
        g.wait()
        if t + 1 < _NCH:
            g_next = gather_chunk(t + 1)
        if writes[t % 2] is not None:
            writes[t % 2].wait()
        transpose_chunk(t)
        writes[t % 2] = write_chunk(t)
        if t + 1 < _NCH:
            g = g_next
    writes[(_NCH - 1) % 2].wait()
    writes[_NCH % 2].wait()


@jax.jit
def kernel(labels, table):
    table_pad = jnp.pad(table, ((0, 0), (0, PAD_DIM - EMBED_DIM)))
    k = functools.partial(
        pl.kernel,
        mesh=plsc.VectorSubcoreMesh(core_axis_name="c", subcore_axis_name="s"),
        out_type=jax.ShapeDtypeStruct((EMBED_DIM, BATCH), jnp.float32),
        scratch_types=[
            pltpu.VMEM((_B_PER_W,), jnp.int32),
            pltpu.VMEM((2, _CHUNK, PAD_DIM), jnp.float32),
            pltpu.VMEM((2, EMBED_DIM, _CHUNK), jnp.float32),
            [pltpu.SemaphoreType.DMA, pltpu.SemaphoreType.DMA],
            [pltpu.SemaphoreType.DMA, pltpu.SemaphoreType.DMA],
        ],
        compiler_params=pltpu.CompilerParams(
            use_tc_tiling_on_sc=True, needs_layout_passes=False
        ),
    )(_lookup_body)
    return k(labels, table_pad).T


# speedup vs baseline: 2.0244x; 1.5911x over previous
import functools

import jax
import jax.numpy as jnp
from jax import lax
from jax.experimental import pallas as pl
from jax.experimental.pallas import tpu as pltpu
from jax.experimental.pallas import tpu_sc as plsc

BATCH = 16384
EMBED_DIM = 64
VOCAB_ROWS = 1000
LANES = 16

_info = plsc.get_sparse_core_info()
_NC, _NS = _info.num_cores, _info.num_subcores
_NW = _NC * _NS
_JG = 8
_NQ = _NW // (EMBED_DIM // _JG)
_B_PER_Q = BATCH // _NQ
_CHUNK = 512
_NCH = _B_PER_Q // _CHUNK
_VPC = _JG * _CHUNK // LANES


def _lookup_body(labels_hbm, tablet_hbm, out_hbm, idx_v, tbl_v, t_v, wsem):
    wid = lax.axis_index("s") * _NC + lax.axis_index("c")
    g = wid // _NQ
    q = wid % _NQ
    pltpu.sync_copy(tablet_hbm.at[pl.ds(g * _JG, _JG)], tbl_v)
    pltpu.sync_copy(labels_hbm.at[pl.ds(q * _B_PER_Q, _B_PER_Q)], idx_v)

    def compute_chunk(c):
        @plsc.parallel_loop(0, _VPC, unroll=8)
        def _(v):
            j = lax.shift_right_logical(v, 5)
            blk = lax.bitwise_and(v, 31)
            idx_vec = idx_v[pl.ds(c * _CHUNK + blk * LANES, LANES)]
            j_vec = jnp.full((LANES,), j, jnp.int32)
            vals = plsc.load_gather(tbl_v, [j_vec, idx_vec])
            t_v[c % 2, j, pl.ds(blk * LANES, LANES)] = vals

    def write_chunk(c):
        return pltpu.async_copy(
            t_v.at[c % 2],
            out_hbm.at[
                pl.ds(g * _JG, _JG), pl.ds(q * _B_PER_Q + c * _CHUNK, _CHUNK)
            ],
            wsem[c % 2],
        )

    writes = [None, None]
    for c in range(_NCH):
        if writes[c % 2] is not None:
            writes[c % 2].wait()
        compute_chunk(c)
        writes[c % 2] = write_chunk(c)
    writes[0].wait()
    writes[1].wait()


@jax.jit
def kernel(labels, table):
    k = functools.partial(
        pl.kernel,
        mesh=plsc.VectorSubcoreMesh(core_axis_name="c", subcore_axis_name="s"),
        out_type=jax.ShapeDtypeStruct((EMBED_DIM, BATCH), jnp.float32),
        scratch_types=[
            pltpu.VMEM((_B_PER_Q,), jnp.int32),
            pltpu.VMEM((_JG, VOCAB_ROWS), jnp.float32),
            pltpu.VMEM((2, _JG, _CHUNK), jnp.float32),
            [pltpu.SemaphoreType.DMA, pltpu.SemaphoreType.DMA],
        ],
        compiler_params=pltpu.CompilerParams(
            use_tc_tiling_on_sc=True, needs_layout_passes=False
        ),
    )(_lookup_body)
    return k(labels, table.T).T
